# baseline (speedup 1.0000x reference)
import jax
import jax.numpy as jnp
import numpy as np
from jax.experimental import pallas as pl
from jax.experimental.pallas import tpu as pltpu
from jax.sharding import Mesh, PartitionSpec as P

try:
    from jax.experimental.shard_map import shard_map
except ImportError:
    from jax.shard_map import shard_map

_E = 8
_R = 16
_ALPHA = 2.0
_TB = 256


def _moe_lora_kernel(x_ref, wr_ref, wup_ref, wgate_ref, wdown_ref,
                     b1c_ref, a1_ref, b3c_ref, a3_ref, b2_ref, a2c_ref,
                     out_ref):
    f32 = jnp.float32
    bf16 = jnp.bfloat16
    x = x_ref[...]
    xb = x.astype(bf16)

    logits = jnp.dot(xb, wr_ref[...], preferred_element_type=f32)
    idx = jax.lax.broadcasted_iota(jnp.int32, logits.shape, 1)
    m1 = jnp.max(logits, axis=1, keepdims=True)
    i1 = jnp.min(jnp.where(logits == m1, idx, _E), axis=1, keepdims=True)
    masked = jnp.where(idx == i1, -jnp.inf, logits)
    m2 = jnp.max(masked, axis=1, keepdims=True)
    i2 = jnp.min(jnp.where(masked == m2, idx, _E), axis=1, keepdims=True)
    w1 = jax.nn.sigmoid(m1 - m2)
    w2 = 1.0 - w1

    U = jnp.dot(xb, wup_ref[...], preferred_element_type=f32)
    G = jnp.dot(xb, wgate_ref[...], preferred_element_type=f32)
    XB1 = jnp.dot(xb, b1c_ref[...], preferred_element_type=f32)
    XB3 = jnp.dot(xb, b3c_ref[...], preferred_element_type=f32)

    H = jnp.zeros(U.shape, f32)
    qs = []
    for e in range(_E):
        p1 = jnp.dot(XB1[:, e * _R:(e + 1) * _R].astype(bf16), a1_ref[e],
                     preferred_element_type=f32)
        p3 = jnp.dot(XB3[:, e * _R:(e + 1) * _R].astype(bf16), a3_ref[e],
                     preferred_element_type=f32)
        h = jax.nn.silu(U + p1) * (G + p3)
        w_e = jnp.where(i1 == e, w1, 0.0) + jnp.where(i2 == e, w2, 0.0)
        hw = h * w_e
        H = H + hw
        qs.append(jnp.dot(hw.astype(bf16), b2_ref[e],
                          preferred_element_type=f32))
    Q = jnp.concatenate(qs, axis=1)
    out = jnp.dot(H.astype(bf16), wdown_ref[...], preferred_element_type=f32)
    out = out + jnp.dot(Q.astype(bf16), a2c_ref[...],
                        preferred_element_type=f32)
    out_ref[...] = out


def _ffn(x, wr, wup, wgate, wdown, b1c, a1t, b3c, a3t, b2t, a2c):
    T, D = x.shape
    F = wup.shape[1]
    grid = (T // _TB,)
    return pl.pallas_call(
        _moe_lora_kernel,
        grid=grid,
        in_specs=[
            pl.BlockSpec((_TB, D), lambda i: (i, 0)),
            pl.BlockSpec((D, _E), lambda i: (0, 0)),
            pl.BlockSpec((D, F), lambda i: (0, 0)),
            pl.BlockSpec((D, F), lambda i: (0, 0)),
            pl.BlockSpec((F, D), lambda i: (0, 0)),
            pl.BlockSpec((D, _E * _R), lambda i: (0, 0)),
            pl.BlockSpec((_E, _R, F), lambda i: (0, 0, 0)),
            pl.BlockSpec((D, _E * _R), lambda i: (0, 0)),
            pl.BlockSpec((_E, _R, F), lambda i: (0, 0, 0)),
            pl.BlockSpec((_E, F, _R), lambda i: (0, 0, 0)),
            pl.BlockSpec((_E * _R, D), lambda i: (0, 0)),
        ],
        out_specs=pl.BlockSpec((_TB, D), lambda i: (i, 0)),
        out_shape=jax.ShapeDtypeStruct((T, D), jnp.float32),
    )(x, wr, wup, wgate, wdown, b1c, a1t, b3c, a3t, b2t, a2c)


def kernel(x, W_up, W_gate_proj, W_down, W_router, A1, B1, A2, B2, A3, B3):
    T, D = x.shape
    F = W_up.shape[0]
    bf16 = jnp.bfloat16
    wr = W_router.T.astype(bf16)
    wup = W_up.T.astype(bf16)
    wgate = W_gate_proj.T.astype(bf16)
    wdown = W_down.T.astype(bf16)
    b1c = B1.transpose(2, 0, 1).reshape(F, _E * _R).astype(bf16)
    b3c = B3.transpose(2, 0, 1).reshape(D, _E * _R).astype(bf16)
    a1t = (_ALPHA * A1.transpose(0, 2, 1)).astype(bf16)
    a3t = (_ALPHA * A3.transpose(0, 2, 1)).astype(bf16)
    b2t = B2.transpose(0, 2, 1).astype(bf16)
    a2c = (_ALPHA * A2.transpose(0, 2, 1)).reshape(_E * _R, D).astype(bf16)

    args = (x, wr, wup, wgate, wdown, b1c, a1t, b3c, a3t, b2t, a2c)
    devs = jax.devices()
    n_dev = 2 if (len(devs) >= 2 and T % (2 * _TB) == 0) else 1
    if n_dev == 1:
        return _ffn(*args)
    mesh = Mesh(np.array(devs[:2]), ("d",))
    rep = lambda a: P(*((None,) * a.ndim))
    f = shard_map(
        _ffn, mesh=mesh,
        in_specs=(P("d", None),) + tuple(rep(a) for a in args[1:]),
        out_specs=P("d", None),
        check_rep=False)
    return f(*args)


# speedup vs baseline: 1.2663x; 1.2663x over previous
import jax
import jax.numpy as jnp
from jax.experimental import pallas as pl
from jax.experimental.pallas import tpu as pltpu

_E = 8
_R = 16
_ALPHA = 2.0
_TB = 256


def _moe_lora_kernel(x_ref, wr_ref, wup_ref, wgate_ref, wdown_ref,
                     b1c_ref, a1_ref, b3c_ref, a3_ref, b2_ref, a2c_ref,
                     out_ref):
    f32 = jnp.float32
    bf16 = jnp.bfloat16
    xb = x_ref[...]

    logits = jnp.dot(xb, wr_ref[...], preferred_element_type=f32)
    idx = jax.lax.broadcasted_iota(jnp.int32, logits.shape, 1)
    m1 = jnp.max(logits, axis=1, keepdims=True)
    i1 = jnp.min(jnp.where(logits == m1, idx, _E), axis=1, keepdims=True)
    masked = jnp.where(idx == i1, -jnp.inf, logits)
    m2 = jnp.max(masked, axis=1, keepdims=True)
    i2 = jnp.min(jnp.where(masked == m2, idx, _E), axis=1, keepdims=True)
    w1 = jax.nn.sigmoid(m1 - m2)
    w2 = 1.0 - w1

    U = jnp.dot(xb, wup_ref[...], preferred_element_type=f32)
    G = jnp.dot(xb, wgate_ref[...], preferred_element_type=f32)
    XB1 = jnp.dot(xb, b1c_ref[...], preferred_element_type=f32)
    XB3 = jnp.dot(xb, b3c_ref[...], preferred_element_type=f32)
    XB1 = XB1.astype(bf16)
    XB3 = XB3.astype(bf16)

    H = jnp.zeros(U.shape, f32)
    qs = []
    for e in range(_E):
        p1 = jnp.dot(XB1[:, e * _R:(e + 1) * _R], a1_ref[e],
                     preferred_element_type=f32)
        p3 = jnp.dot(XB3[:, e * _R:(e + 1) * _R], a3_ref[e],
                     preferred_element_type=f32)
        z = (U + p1).astype(bf16)
        g = (G + p3).astype(bf16)
        h = (z * jax.nn.sigmoid(z)) * g
        w_e = (jnp.where(i1 == e, w1, 0.0)
               + jnp.where(i2 == e, w2, 0.0)).astype(bf16)
        hw = h * w_e
        H = H + hw.astype(f32)
        qs.append(jnp.dot(hw, b2_ref[e], preferred_element_type=f32))
    Q = jnp.concatenate(qs, axis=1)
    out = jnp.dot(H.astype(bf16), wdown_ref[...], preferred_element_type=f32)
    out = out + jnp.dot(Q.astype(bf16), a2c_ref[...],
                        preferred_element_type=f32)
    out_ref[...] = out


def kernel(x, W_up, W_gate_proj, W_down, W_router, A1, B1, A2, B2, A3, B3):
    T, D = x.shape
    F = W_up.shape[0]
    bf16 = jnp.bfloat16
    xb = x.astype(bf16)
    wr = W_router.T.astype(bf16)
    wup = W_up.T.astype(bf16)
    wgate = W_gate_proj.T.astype(bf16)
    wdown = W_down.T.astype(bf16)
    b1c = B1.transpose(2, 0, 1).reshape(F, _E * _R).astype(bf16)
    b3c = B3.transpose(2, 0, 1).reshape(D, _E * _R).astype(bf16)
    a1t = (_ALPHA * A1.transpose(0, 2, 1)).astype(bf16)
    a3t = (_ALPHA * A3.transpose(0, 2, 1)).astype(bf16)
    b2t = B2.transpose(0, 2, 1).astype(bf16)
    a2c = (_ALPHA * A2.transpose(0, 2, 1)).reshape(_E * _R, D).astype(bf16)

    grid = (T // _TB,)
    out = pl.pallas_call(
        _moe_lora_kernel,
        grid=grid,
        in_specs=[
            pl.BlockSpec((_TB, D), lambda i: (i, 0)),
            pl.BlockSpec((D, _E), lambda i: (0, 0)),
            pl.BlockSpec((D, F), lambda i: (0, 0)),
            pl.BlockSpec((D, F), lambda i: (0, 0)),
            pl.BlockSpec((F, D), lambda i: (0, 0)),
            pl.BlockSpec((D, _E * _R), lambda i: (0, 0)),
            pl.BlockSpec((_E, _R, F), lambda i: (0, 0, 0)),
            pl.BlockSpec((D, _E * _R), lambda i: (0, 0)),
            pl.BlockSpec((_E, _R, F), lambda i: (0, 0, 0)),
            pl.BlockSpec((_E, F, _R), lambda i: (0, 0, 0)),
            pl.BlockSpec((_E * _R, D), lambda i: (0, 0)),
        ],
        out_specs=pl.BlockSpec((_TB, D), lambda i: (i, 0)),
        out_shape=jax.ShapeDtypeStruct((T, D), jnp.float32),
    )(xb, wr, wup, wgate, wdown, b1c, a1t, b3c, a3t, b2t, a2c)
    return out
